# baseline (speedup 1.0000x reference)
import functools

import jax
import jax.numpy as jnp
from jax import lax
from jax.experimental import pallas as pl
from jax.experimental.pallas import tpu as pltpu
from jax.experimental.pallas import tpu_sc as plsc

B = 16384
D = 64
L = 16
NC = 2
NS = 16
NW = NC * NS
BPW = B // NW
CHUNK = 128
NCHUNK = BPW // CHUNK


def _transe_sc(ph_hbm, pr_hbm, pt_hbm, nh_hbm, nt_hbm, ent_hbm, rel_hbm,
               pos_hbm, neg_hbm,
               phv, prv, ptv, nhv, ntv,
               ph_rows, pt_rows, nh_rows, nt_rows, r_rows,
               pos_v, neg_v, sem):
    wid = lax.axis_index("s") * NC + lax.axis_index("c")
    lane = lax.iota(jnp.int32, L)

    def chunk_body(c, chunk_carry):
        base = wid * BPW + c * CHUNK
        sl = pl.ds(base, CHUNK)
        pltpu.sync_copy(ph_hbm.at[sl], phv)
        pltpu.sync_copy(pr_hbm.at[sl], prv)
        pltpu.sync_copy(pt_hbm.at[sl], ptv)
        pltpu.sync_copy(nh_hbm.at[sl], nhv)
        pltpu.sync_copy(nt_hbm.at[sl], ntv)
        g1 = pltpu.async_copy(ent_hbm.at[phv], ph_rows, sem)
        g2 = pltpu.async_copy(ent_hbm.at[ptv], pt_rows, sem)
        g3 = pltpu.async_copy(ent_hbm.at[nhv], nh_rows, sem)
        g4 = pltpu.async_copy(ent_hbm.at[ntv], nt_rows, sem)
        g5 = pltpu.async_copy(rel_hbm.at[prv], r_rows, sem)
        g1.wait(); g2.wait(); g3.wait(); g4.wait(); g5.wait()

        def body(g, carry):
            rowidx = g * L + lane
            pacc = jnp.zeros((L,), jnp.float32)
            nacc = jnp.zeros((L,), jnp.float32)
            for d in range(D):
                dvec = (lane + d) & (D - 1)
                r = plsc.load_gather(r_rows, [rowidx, dvec])
                ph = plsc.load_gather(ph_rows, [rowidx, dvec])
                pt = plsc.load_gather(pt_rows, [rowidx, dvec])
                nh = plsc.load_gather(nh_rows, [rowidx, dvec])
                nt = plsc.load_gather(nt_rows, [rowidx, dvec])
                pacc = pacc + jnp.abs(ph + r - pt)
                nacc = nacc + jnp.abs(nh + r - nt)
            pos_v[pl.ds(g * L, L)] = pacc
            neg_v[pl.ds(g * L, L)] = nacc
            return carry

        lax.fori_loop(0, CHUNK // L, body, 0)
        pltpu.sync_copy(pos_v, pos_hbm.at[sl])
        pltpu.sync_copy(neg_v, neg_hbm.at[sl])
        return chunk_carry

    lax.fori_loop(0, NCHUNK, chunk_body, 0)


@jax.jit
def kernel(pos_samples, neg_samples, entity_table, relation_table):
    ph = pos_samples[:, 0].astype(jnp.int32)
    pr = pos_samples[:, 1].astype(jnp.int32)
    pt = pos_samples[:, 2].astype(jnp.int32)
    nh = neg_samples[:, 0].astype(jnp.int32)
    nt = neg_samples[:, 2].astype(jnp.int32)

    mesh = plsc.VectorSubcoreMesh(core_axis_name="c", subcore_axis_name="s")
    f = pl.kernel(
        _transe_sc,
        out_type=(
            jax.ShapeDtypeStruct((B,), jnp.float32),
            jax.ShapeDtypeStruct((B,), jnp.float32),
        ),
        mesh=mesh,
        compiler_params=pltpu.CompilerParams(
            needs_layout_passes=False, use_tc_tiling_on_sc=False),
        scratch_types=[
            pltpu.VMEM((CHUNK,), jnp.int32),
            pltpu.VMEM((CHUNK,), jnp.int32),
            pltpu.VMEM((CHUNK,), jnp.int32),
            pltpu.VMEM((CHUNK,), jnp.int32),
            pltpu.VMEM((CHUNK,), jnp.int32),
            pltpu.VMEM((CHUNK, D), jnp.float32),
            pltpu.VMEM((CHUNK, D), jnp.float32),
            pltpu.VMEM((CHUNK, D), jnp.float32),
            pltpu.VMEM((CHUNK, D), jnp.float32),
            pltpu.VMEM((CHUNK, D), jnp.float32),
            pltpu.VMEM((CHUNK,), jnp.float32),
            pltpu.VMEM((CHUNK,), jnp.float32),
            pltpu.SemaphoreType.DMA,
        ],
    )
    return f(ph, pr, pt, nh, nt, entity_table, relation_table)
